# baseline (device time: 51087 ns/iter reference)
import jax
import jax.numpy as jnp
from jax import lax
from jax.experimental import pallas as pl
from jax.experimental.pallas import tpu as pltpu

N_DEV = 4


def kernel(t, W):
    m_per, k = t.shape
    _, n = W.shape
    ch = m_per // N_DEV

    def body(t_ref, w_ref, out_ref, send_buf, recv_buf, send_sems, recv_sems):
        my_pos = lax.axis_index("i")
        left = (my_pos - 1) % N_DEV
        right = (my_pos + 1) % N_DEV

        barrier_sem = pltpu.get_barrier_semaphore()
        for nbr in (left, right):
            pl.semaphore_signal(
                barrier_sem, inc=1,
                device_id=(nbr,), device_id_type=pl.DeviceIdType.MESH,
            )
        pl.semaphore_wait(barrier_sem, 2)

        def rows(c):
            return pl.ds(c * ch, ch)

        send_buf[0, :, :] = t_ref[rows(my_pos), :]
        for h in range(N_DEV - 1):
            rdma = pltpu.make_async_remote_copy(
                src_ref=send_buf.at[h],
                dst_ref=recv_buf.at[h],
                send_sem=send_sems.at[h],
                recv_sem=recv_sems.at[h],
                device_id=(right,),
                device_id_type=pl.DeviceIdType.MESH,
            )
            rdma.start()
            rdma.wait()
            if h < N_DEV - 2:
                c_next = (my_pos - h - 1) % N_DEV
                send_buf[h + 1, :, :] = (
                    recv_buf[h, :, :] + t_ref[rows(c_next), :]
                )

        c_own = (my_pos + 1) % N_DEV
        s_own = recv_buf[N_DEV - 2, :, :] + t_ref[rows(c_own), :]
        out_ref[rows(c_own), :] = jnp.dot(
            s_own, w_ref[:, :], preferred_element_type=jnp.float32
        )

        for g in range(N_DEV - 1):
            c_send = (my_pos + 1 - g) % N_DEV
            rdma = pltpu.make_async_remote_copy(
                src_ref=out_ref.at[rows(c_send), :],
                dst_ref=out_ref.at[rows(c_send), :],
                send_sem=send_sems.at[N_DEV - 1 + g],
                recv_sem=recv_sems.at[N_DEV - 1 + g],
                device_id=(right,),
                device_id_type=pl.DeviceIdType.MESH,
            )
            rdma.start()
            rdma.wait()

    return pl.pallas_call(
        body,
        out_shape=jax.ShapeDtypeStruct((m_per, n), jnp.float32),
        in_specs=[
            pl.BlockSpec(memory_space=pltpu.VMEM),
            pl.BlockSpec(memory_space=pltpu.VMEM),
        ],
        out_specs=pl.BlockSpec(memory_space=pltpu.VMEM),
        scratch_shapes=[
            pltpu.VMEM((N_DEV - 1, ch, k), jnp.float32),
            pltpu.VMEM((N_DEV - 1, ch, k), jnp.float32),
            pltpu.SemaphoreType.DMA((2 * (N_DEV - 1),)),
            pltpu.SemaphoreType.DMA((2 * (N_DEV - 1),)),
        ],
        compiler_params=pltpu.CompilerParams(collective_id=0),
    )(t, W)


# device time: 30997 ns/iter; 1.6481x vs baseline; 1.6481x over previous
import jax
import jax.numpy as jnp
from jax import lax
from jax.experimental import pallas as pl
from jax.experimental.pallas import tpu as pltpu

N_DEV = 4


def kernel(t, W):
    m_per, k = t.shape
    _, n = W.shape
    H = m_per // 2
    Q = H // 2
    E = Q // 2

    def body(t_ref, w_ref, out_ref,
             red_a, red_b, recv_a1, recv_b1, recv_a2, recv_b2,
             send_sems, recv_sems):
        p = lax.axis_index("i")
        m1 = 3 - p
        m2 = p ^ 1
        k1 = p // 2
        k2 = p % 2
        j1 = (k1 + k2) % 2
        j2 = k1

        barrier_sem = pltpu.get_barrier_semaphore()
        for nbr in (m1, m2):
            pl.semaphore_signal(
                barrier_sem, inc=1,
                device_id=(nbr,), device_id_type=pl.DeviceIdType.MESH,
            )
        pl.semaphore_wait(barrier_sem, 2)

        def xch(idx, src, dst, partner):
            rdma = pltpu.make_async_remote_copy(
                src_ref=src, dst_ref=dst,
                send_sem=send_sems.at[idx], recv_sem=recv_sems.at[idx],
                device_id=(partner,), device_id_type=pl.DeviceIdType.MESH,
            )
            rdma.start()
            return rdma

        r_a1 = xch(0, t_ref.at[pl.ds((1 - k1) * Q, Q), :], recv_a1, m1)
        r_b1 = xch(1, t_ref.at[pl.ds(H + (1 - j1) * Q, Q), :], recv_b1, m2)
        r_a1.wait()
        r_b1.wait()
        red_a[:, :] = t_ref[pl.ds(k1 * Q, Q), :] + recv_a1[:, :]
        red_b[:, :] = t_ref[pl.ds(H + j1 * Q, Q), :] + recv_b1[:, :]

        r_a2 = xch(2, red_a.at[pl.ds((1 - k2) * E, E), :], recv_a2, m2)
        r_b2 = xch(3, red_b.at[pl.ds((1 - j2) * E, E), :], recv_b2, m1)
        r_a2.wait()
        r_b2.wait()

        row_a = k1 * Q + k2 * E
        row_b = H + j1 * Q + j2 * E
        s_a = red_a[pl.ds(k2 * E, E), :] + recv_a2[:, :]
        s_b = red_b[pl.ds(j2 * E, E), :] + recv_b2[:, :]
        w = w_ref[:, :]
        out_ref[pl.ds(row_a, E), :] = jnp.dot(
            s_a, w, preferred_element_type=jnp.float32)
        out_ref[pl.ds(row_b, E), :] = jnp.dot(
            s_b, w, preferred_element_type=jnp.float32)

        r_a3 = xch(4, out_ref.at[pl.ds(row_a, E), :],
                   out_ref.at[pl.ds(row_a, E), :], m2)
        r_b3 = xch(5, out_ref.at[pl.ds(row_b, E), :],
                   out_ref.at[pl.ds(row_b, E), :], m1)
        r_a3.wait()
        r_b3.wait()

        r_a4 = xch(6, out_ref.at[pl.ds(k1 * Q, Q), :],
                   out_ref.at[pl.ds(k1 * Q, Q), :], m1)
        r_b4 = xch(7, out_ref.at[pl.ds(H + j1 * Q, Q), :],
                   out_ref.at[pl.ds(H + j1 * Q, Q), :], m2)
        r_a4.wait()
        r_b4.wait()

    return pl.pallas_call(
        body,
        out_shape=jax.ShapeDtypeStruct((m_per, n), jnp.float32),
        in_specs=[
            pl.BlockSpec(memory_space=pltpu.VMEM),
            pl.BlockSpec(memory_space=pltpu.VMEM),
        ],
        out_specs=pl.BlockSpec(memory_space=pltpu.VMEM),
        scratch_shapes=[
            pltpu.VMEM((Q, k), jnp.float32),
            pltpu.VMEM((Q, k), jnp.float32),
            pltpu.VMEM((Q, k), jnp.float32),
            pltpu.VMEM((Q, k), jnp.float32),
            pltpu.VMEM((E, k), jnp.float32),
            pltpu.VMEM((E, k), jnp.float32),
            pltpu.SemaphoreType.DMA((8,)),
            pltpu.SemaphoreType.DMA((8,)),
        ],
        compiler_params=pltpu.CompilerParams(collective_id=0),
    )(t, W)


# device time: 27947 ns/iter; 1.8280x vs baseline; 1.1091x over previous
import jax
import jax.numpy as jnp
from jax import lax
from jax.experimental import pallas as pl
from jax.experimental.pallas import tpu as pltpu

N_DEV = 4
NC = 2


def kernel(t, W):
    m_per, k = t.shape
    _, n = W.shape
    H = m_per // 2
    Q = H // 2
    E = Q // 2
    KC = k // NC
    OC = n // NC

    def body(t_ref, w_ref, out_ref,
             red_a, red_b, recv_a1, recv_b1, recv_a2, recv_b2,
             send_sems, recv_sems):
        p = lax.axis_index("i")
        m1 = 3 - p
        m2 = p ^ 1
        k1 = p // 2
        k2 = p % 2
        j1 = (k1 + k2) % 2
        j2 = k1

        barrier_sem = pltpu.get_barrier_semaphore()
        for nbr in (m1, m2):
            pl.semaphore_signal(
                barrier_sem, inc=1,
                device_id=(nbr,), device_id_type=pl.DeviceIdType.MESH,
            )
        pl.semaphore_wait(barrier_sem, 2)

        started = []

        def xch(idx, src, dst, partner):
            rdma = pltpu.make_async_remote_copy(
                src_ref=src, dst_ref=dst,
                send_sem=send_sems.at[idx], recv_sem=recv_sems.at[idx],
                device_id=(partner,), device_id_type=pl.DeviceIdType.MESH,
            )
            rdma.start()
            started.append(rdma)
            return rdma

        def kc(c):
            return pl.ds(c * KC, KC)

        def oc(c):
            return pl.ds(c * OC, OC)

        row_a = k1 * Q + k2 * E
        row_b = H + j1 * Q + j2 * E

        r_a1 = [xch(0 + c, t_ref.at[pl.ds((1 - k1) * Q, Q), kc(c)],
                    recv_a1.at[:, kc(c)], m1) for c in range(NC)]
        r_b1 = [xch(2 + c, t_ref.at[pl.ds(H + (1 - j1) * Q, Q), kc(c)],
                    recv_b1.at[:, kc(c)], m2) for c in range(NC)]

        r_a2, r_b2 = [], []
        for c in range(NC):
            r_a1[c].wait_recv()
            red_a[:, kc(c)] = t_ref[pl.ds(k1 * Q, Q), kc(c)] + recv_a1[:, kc(c)]
            r_a2.append(xch(4 + c, red_a.at[pl.ds((1 - k2) * E, E), kc(c)],
                            recv_a2.at[:, kc(c)], m2))
            r_b1[c].wait_recv()
            red_b[:, kc(c)] = t_ref[pl.ds(H + j1 * Q, Q), kc(c)] + recv_b1[:, kc(c)]
            r_b2.append(xch(6 + c, red_b.at[pl.ds((1 - j2) * E, E), kc(c)],
                            recv_b2.at[:, kc(c)], m1))

        for c in range(NC):
            r_a2[c].wait_recv()
            s_a = red_a[pl.ds(k2 * E, E), kc(c)] + recv_a2[:, kc(c)]
            part = jnp.dot(s_a, w_ref[kc(c), :],
                           preferred_element_type=jnp.float32)
            if c == 0:
                out_ref[pl.ds(row_a, E), :] = part
            else:
                out_ref[pl.ds(row_a, E), :] = out_ref[pl.ds(row_a, E), :] + part
            r_b2[c].wait_recv()
            s_b = red_b[pl.ds(j2 * E, E), kc(c)] + recv_b2[:, kc(c)]
            part = jnp.dot(s_b, w_ref[kc(c), :],
                           preferred_element_type=jnp.float32)
            if c == 0:
                out_ref[pl.ds(row_b, E), :] = part
            else:
                out_ref[pl.ds(row_b, E), :] = out_ref[pl.ds(row_b, E), :] + part

        r_a3 = [xch(8 + c, out_ref.at[pl.ds(row_a, E), oc(c)],
                    out_ref.at[pl.ds(row_a, E), oc(c)], m2) for c in range(NC)]
        r_b3 = [xch(10 + c, out_ref.at[pl.ds(row_b, E), oc(c)],
                    out_ref.at[pl.ds(row_b, E), oc(c)], m1) for c in range(NC)]

        r_4 = []
        for c in range(NC):
            r_a3[c].wait_recv()
            r_4.append(xch(12 + c, out_ref.at[pl.ds(k1 * Q, Q), oc(c)],
                           out_ref.at[pl.ds(k1 * Q, Q), oc(c)], m1))
            r_b3[c].wait_recv()
            r_4.append(xch(14 + c, out_ref.at[pl.ds(H + j1 * Q, Q), oc(c)],
                           out_ref.at[pl.ds(H + j1 * Q, Q), oc(c)], m2))

        for r in r_4:
            r.wait_recv()
        for r in started:
            r.wait_send()

    return pl.pallas_call(
        body,
        out_shape=jax.ShapeDtypeStruct((m_per, n), jnp.float32),
        in_specs=[
            pl.BlockSpec(memory_space=pltpu.VMEM),
            pl.BlockSpec(memory_space=pltpu.VMEM),
        ],
        out_specs=pl.BlockSpec(memory_space=pltpu.VMEM),
        scratch_shapes=[
            pltpu.VMEM((Q, k), jnp.float32),
            pltpu.VMEM((Q, k), jnp.float32),
            pltpu.VMEM((Q, k), jnp.float32),
            pltpu.VMEM((Q, k), jnp.float32),
            pltpu.VMEM((E, k), jnp.float32),
            pltpu.VMEM((E, k), jnp.float32),
            pltpu.SemaphoreType.DMA((16,)),
            pltpu.SemaphoreType.DMA((16,)),
        ],
        compiler_params=pltpu.CompilerParams(collective_id=0),
    )(t, W)
